# baseline (speedup 1.0000x reference)
import jax
import jax.numpy as jnp
from jax import lax
from jax.experimental import pallas as pl
from jax.experimental.pallas import tpu as pltpu
from jax.experimental.pallas import tpu_sc as plsc

NUM_EMBEDDINGS = 8192
EMBEDDING_DIM = 64
BATCH = 32
TOKENS = 1024

_NC = 2
_NS = 16
_CHUNK = 128
_NCHUNK = TOKENS // _CHUNK
_F0S = (0, 16, 32, 48)


def _gather_body(table_hbm, idx_hbm, out_hbm, idx_v,
                 rows_a, rows_b, t_a, t_b,
                 gsem_a, gsem_b, ssem_a, ssem_b):
    wid = lax.axis_index("s") * _NC + lax.axis_index("c")
    rh = wid // 8
    rl = wid % 8
    pltpu.sync_copy(idx_hbm.at[rh, :, rl], idx_v)

    rows = [rows_a, rows_b]
    tbuf = [t_a, t_b]
    gsem = [gsem_a, gsem_b]
    ssem = [ssem_a, ssem_b]

    def fire_gather(c):
        return pltpu.async_copy(
            table_hbm.at[idx_v.at[c]], rows[c % 2], gsem[c % 2]
        )

    iota = lax.iota(jnp.int32, 16)
    fh_idx = [(iota + f0) // 8 for f0 in _F0S]
    fl_idx = [(iota + f0) % 8 for f0 in _F0S]

    gathers = [fire_gather(0)]
    stores = [None, None]
    for c in range(_NCHUNK):
        if c + 1 < _NCHUNK:
            gathers.append(fire_gather(c + 1))
        gathers[c].wait()
        if stores[c % 2] is not None:
            stores[c % 2].wait()
        G = rows[c % 2]
        T = tbuf[c % 2]

        @plsc.parallel_loop(0, _CHUNK, step=1, unroll=4)
        def _transpose(t):
            tv = lax.broadcast(t, (16,))
            for i, f0 in enumerate(_F0S):
                v = G[t, pl.ds(f0, 16)]
                plsc.store_scatter(T, [fh_idx[i], fl_idx[i], tv], v)

        stores[c % 2] = pltpu.async_copy(
            T, out_hbm.at[wid, :, c], ssem[c % 2]
        )
    stores[0].wait()
    stores[1].wait()


_gather_call = pl.kernel(
    _gather_body,
    out_type=jax.ShapeDtypeStruct((BATCH, 8, _NCHUNK, 8, _CHUNK), jnp.float32),
    mesh=plsc.VectorSubcoreMesh(core_axis_name="c", subcore_axis_name="s"),
    scratch_types=[
        pltpu.VMEM((_NCHUNK, _CHUNK), jnp.int32),
        pltpu.VMEM((_CHUNK, EMBEDDING_DIM), jnp.float32),
        pltpu.VMEM((_CHUNK, EMBEDDING_DIM), jnp.float32),
        pltpu.VMEM((8, 8, _CHUNK), jnp.float32),
        pltpu.VMEM((8, 8, _CHUNK), jnp.float32),
        pltpu.SemaphoreType.DMA,
        pltpu.SemaphoreType.DMA,
        pltpu.SemaphoreType.DMA,
        pltpu.SemaphoreType.DMA,
    ],
    compiler_params=pltpu.CompilerParams(
        use_tc_tiling_on_sc=False, needs_layout_passes=False
    ),
)


@jax.jit
def kernel(indices, embeddings):
    idx_t = (
        jnp.asarray(indices, jnp.int32)
        .reshape(4, 8, 8, 128)
        .transpose(0, 2, 1, 3)
    )
    out5 = _gather_call(embeddings, idx_t)
    return out5.transpose(0, 2, 4, 1, 3).reshape(BATCH, TOKENS, EMBEDDING_DIM)


# speedup vs baseline: 1.5984x; 1.5984x over previous
import jax
import jax.numpy as jnp
from jax import lax
from jax.experimental import pallas as pl
from jax.experimental.pallas import tpu as pltpu
from jax.experimental.pallas import tpu_sc as plsc

NUM_EMBEDDINGS = 8192
EMBEDDING_DIM = 64
BATCH = 32
TOKENS = 1024

_NC = 2
_NS = 16
_CHUNK = 128
_NCHUNK = TOKENS // _CHUNK
_F0S = (0, 16, 32, 48)
_TPAD = _CHUNK + 9


def _gather_body(table_hbm, idx_hbm, out_hbm, idx_v,
                 rows_a, rows_b, t_a, t_b,
                 gsem_a, gsem_b, ssem_a, ssem_b):
    wid = lax.axis_index("s") * _NC + lax.axis_index("c")
    rh = wid // 8
    rl = wid % 8
    pltpu.sync_copy(idx_hbm.at[rh, :, rl], idx_v)

    rows = [rows_a, rows_b]
    tbuf = [t_a, t_b]
    gsem = [gsem_a, gsem_b]
    ssem = [ssem_a, ssem_b]

    def fire_gather(c):
        return pltpu.async_copy(
            table_hbm.at[idx_v.at[c]], rows[c % 2], gsem[c % 2]
        )

    iota = lax.iota(jnp.int32, 16)
    fh_idx = [(iota + f0) // 8 for f0 in _F0S]
    fl_idx = [(iota + f0) % 8 for f0 in _F0S]

    gathers = [fire_gather(0)]
    stores = [None, None]
    for c in range(_NCHUNK):
        if c + 1 < _NCHUNK:
            gathers.append(fire_gather(c + 1))
        gathers[c].wait()
        if stores[c % 2] is not None:
            stores[c % 2].wait()
        G = rows[c % 2]
        T = tbuf[c % 2]

        @plsc.parallel_loop(0, _CHUNK, step=1, unroll=4)
        def _transpose(t):
            tv = lax.broadcast(t, (16,))
            for i, f0 in enumerate(_F0S):
                v = G[t, pl.ds(f0, 16)]
                plsc.store_scatter(T, [fh_idx[i], fl_idx[i], tv], v)

        stores[c % 2] = pltpu.async_copy(
            T.at[:, :, pl.ds(0, _CHUNK)], out_hbm.at[wid, :, c], ssem[c % 2]
        )
    stores[0].wait()
    stores[1].wait()


_gather_call = pl.kernel(
    _gather_body,
    out_type=jax.ShapeDtypeStruct((BATCH, 8, _NCHUNK, 8, _CHUNK), jnp.float32),
    mesh=plsc.VectorSubcoreMesh(core_axis_name="c", subcore_axis_name="s"),
    scratch_types=[
        pltpu.VMEM((_NCHUNK, _CHUNK), jnp.int32),
        pltpu.VMEM((_CHUNK, EMBEDDING_DIM), jnp.float32),
        pltpu.VMEM((_CHUNK, EMBEDDING_DIM), jnp.float32),
        pltpu.VMEM((8, 8, _TPAD), jnp.float32),
        pltpu.VMEM((8, 8, _TPAD), jnp.float32),
        pltpu.SemaphoreType.DMA,
        pltpu.SemaphoreType.DMA,
        pltpu.SemaphoreType.DMA,
        pltpu.SemaphoreType.DMA,
    ],
    compiler_params=pltpu.CompilerParams(
        use_tc_tiling_on_sc=False, needs_layout_passes=False
    ),
)


@jax.jit
def kernel(indices, embeddings):
    idx_t = (
        jnp.asarray(indices, jnp.int32)
        .reshape(4, 8, 8, 128)
        .transpose(0, 2, 1, 3)
    )
    out5 = _gather_call(embeddings, idx_t)
    return out5.transpose(0, 2, 4, 1, 3).reshape(BATCH, TOKENS, EMBEDDING_DIM)
